# baseline (speedup 1.0000x reference)
import functools

import jax
import jax.numpy as jnp
from jax import lax
from jax.experimental import pallas as pl
from jax.experimental.pallas import tpu as pltpu
from jax.experimental.pallas import tpu_sc as plsc

TOPK = 8
SEASON = (1, 24)
NEG_INF = float("-inf")


def _rownorm(x):
    n2 = jnp.sum(x * x, axis=-1, keepdims=True)
    return x / jnp.maximum(jnp.sqrt(n2), 1e-12)


def _topk_idx_body(xk_ref, o_ref, *, tm, l, topk, season):
    k = _rownorm(xk_ref[0])
    q = k
    sim = jax.lax.dot_general(
        q, k, (((1,), (1,)), ((), ())), preferred_element_type=jnp.float32)

    colf = jax.lax.broadcasted_iota(jnp.int32, (tm, l), 1).astype(jnp.float32)

    simw = sim
    cols = []
    for _ in range(topk):
        m = jnp.max(simw, axis=1, keepdims=True)
        cand = jnp.where(simw == m, colf, float(l))
        j0 = jnp.min(cand, axis=1, keepdims=True)
        simw = jnp.where(cand == j0, NEG_INF, simw)
        cols.append(j0.astype(jnp.int32))

    i = pl.program_id(1)
    r = i * tm + jax.lax.broadcasted_iota(jnp.int32, (tm, 1), 0)
    deltas = sorted({0, *season, *(-s for s in season)})
    for dlt in deltas:
        c = r + dlt
        cols.append(jnp.where((c >= 0) & (c < l), c, r))
    while len(cols) < 16:
        cols.append(r)
    o_ref[0] = jnp.concatenate(cols, axis=1)


def _make_sc_scatter(nrows, l):
    info = plsc.get_sparse_core_info()
    nworkers = info.num_cores * info.num_subcores
    ch = 16
    rows_per_worker = nrows // nworkers
    nchunk = rows_per_worker // ch
    mesh = plsc.VectorSubcoreMesh(core_axis_name="c", subcore_axis_name="s")

    nbuf = 3
    @functools.partial(
        pl.kernel, mesh=mesh,
        compiler_params=pltpu.CompilerParams(needs_layout_passes=False),
        out_type=jax.ShapeDtypeStruct((nrows, l), jnp.float32),
        scratch_types=[
            *[pltpu.VMEM((ch, l), jnp.float32) for _ in range(nbuf)],
            pltpu.VMEM((rows_per_worker, 16), jnp.int32),
            *[pltpu.SemaphoreType.DMA for _ in range(nbuf)],
        ])
    def sc_scatter(neg_hbm, idx_hbm, out_hbm, *refs):
        bufs, (idx_all,), sems = refs[:nbuf], refs[nbuf:nbuf + 1], refs[nbuf + 1:]
        wid = lax.axis_index("s") * info.num_cores + lax.axis_index("c")
        base = wid * rows_per_worker

        neg = jnp.full((16,), NEG_INF, dtype=jnp.float32)
        zero = jnp.zeros((16,), dtype=jnp.float32)

        pltpu.sync_copy(idx_hbm.at[pl.ds(base, rows_per_worker)], idx_all)
        for bf in bufs:
            pltpu.sync_copy(neg_hbm, bf)

        dmas = [None] * nchunk
        for c in range(nchunk):
            p = c % nbuf
            buf, sem = bufs[p], sems[p]
            row0 = base + c * ch
            if c >= nbuf:
                dmas[c - nbuf].wait()
                for rr in range(ch):
                    plsc.store_scatter(
                        buf,
                        [jnp.full((16,), rr, jnp.int32),
                         idx_all[(c - nbuf) * ch + rr, :]],
                        neg)
            for rr in range(ch):
                plsc.store_scatter(
                    buf,
                    [jnp.full((16,), rr, jnp.int32), idx_all[c * ch + rr, :]],
                    zero)
            dmas[c] = pltpu.async_copy(buf, out_hbm.at[pl.ds(row0, ch)], sem)
        for c in range(nchunk - nbuf, nchunk):
            dmas[c].wait()

    return sc_scatter


@jax.jit
def kernel(tokens):
    b, l, d = tokens.shape
    tm = l

    body = functools.partial(_topk_idx_body, tm=tm, l=l, topk=min(TOPK, l),
                             season=SEASON)
    idx16 = pl.pallas_call(
        body,
        grid=(b, l // tm),
        in_specs=[pl.BlockSpec((1, l, d), lambda bi, i: (bi, 0, 0))],
        out_specs=pl.BlockSpec((1, tm, 16), lambda bi, i: (bi, i, 0)),
        out_shape=jax.ShapeDtypeStruct((b, l, 16), jnp.int32),
        compiler_params=pltpu.CompilerParams(
            dimension_semantics=("parallel", "parallel")),
    )(tokens)

    sc_scatter = _make_sc_scatter(b * l, l)
    neg_chunk = jnp.full((16, l), NEG_INF, dtype=jnp.float32)
    out = sc_scatter(neg_chunk, idx16.reshape(b * l, 16))
    return out.reshape(b, 1, l, l)


# speedup vs baseline: 1.2663x; 1.2663x over previous
import functools

import jax
import jax.numpy as jnp
from jax import lax
from jax.experimental import pallas as pl
from jax.experimental.pallas import tpu as pltpu
from jax.experimental.pallas import tpu_sc as plsc

TOPK = 8
SEASON = (1, 24)
NEG_INF = float("-inf")


def _rownorm(x):
    n2 = jnp.sum(x * x, axis=-1, keepdims=True)
    return x / jnp.maximum(jnp.sqrt(n2), 1e-12)


def _topk_idx_body(xk_ref, o_ref, *, tm, l, topk, season):
    assert l == 2 * tm
    k = _rownorm(xk_ref[0])
    first = jnp.full((tm, 1), pl.program_id(1) == 0)
    q = jnp.where(first, k[:tm], k[tm:])
    sim = jax.lax.dot_general(
        q, k, (((1,), (1,)), ((), ())), preferred_element_type=jnp.float32)

    colf = jax.lax.broadcasted_iota(jnp.int32, (tm, l), 1).astype(jnp.float32)

    simw = sim
    cols = []
    for _ in range(topk):
        m = jnp.max(simw, axis=1, keepdims=True)
        cand = jnp.where(simw == m, colf, float(l))
        j0 = jnp.min(cand, axis=1, keepdims=True)
        simw = jnp.where(cand == j0, NEG_INF, simw)
        cols.append(j0.astype(jnp.int32))

    i = pl.program_id(1)
    r = i * tm + jax.lax.broadcasted_iota(jnp.int32, (tm, 1), 0)
    deltas = sorted({0, *season, *(-s for s in season)})
    for dlt in deltas:
        c = r + dlt
        cols.append(jnp.where((c >= 0) & (c < l), c, r))
    while len(cols) < 16:
        cols.append(r)
    o_ref[0] = jnp.concatenate(cols, axis=1)


def _make_sc_scatter(nrows, l):
    info = plsc.get_sparse_core_info()
    nworkers = info.num_cores * info.num_subcores
    ch = 16
    rows_per_worker = nrows // nworkers
    nchunk = rows_per_worker // ch
    mesh = plsc.VectorSubcoreMesh(core_axis_name="c", subcore_axis_name="s")

    nbuf = 3
    @functools.partial(
        pl.kernel, mesh=mesh,
        compiler_params=pltpu.CompilerParams(needs_layout_passes=False),
        out_type=jax.ShapeDtypeStruct((nrows, l), jnp.float32),
        scratch_types=[
            *[pltpu.VMEM((ch, l), jnp.float32) for _ in range(nbuf)],
            pltpu.VMEM((rows_per_worker, 16), jnp.int32),
            *[pltpu.SemaphoreType.DMA for _ in range(nbuf)],
        ])
    def sc_scatter(neg_hbm, idx_hbm, out_hbm, *refs):
        bufs, (idx_all,), sems = refs[:nbuf], refs[nbuf:nbuf + 1], refs[nbuf + 1:]
        wid = lax.axis_index("s") * info.num_cores + lax.axis_index("c")
        base = wid * rows_per_worker

        neg = jnp.full((16,), NEG_INF, dtype=jnp.float32)
        zero = jnp.zeros((16,), dtype=jnp.float32)

        pltpu.sync_copy(idx_hbm.at[pl.ds(base, rows_per_worker)], idx_all)
        for bf in bufs:
            pltpu.sync_copy(neg_hbm, bf)

        dmas = [None] * nchunk
        for c in range(nchunk):
            p = c % nbuf
            buf, sem = bufs[p], sems[p]
            row0 = base + c * ch
            if c >= nbuf:
                dmas[c - nbuf].wait()
                for rr in range(ch):
                    plsc.store_scatter(
                        buf,
                        [jnp.full((16,), rr, jnp.int32),
                         idx_all[(c - nbuf) * ch + rr, :]],
                        neg)
            for rr in range(ch):
                plsc.store_scatter(
                    buf,
                    [jnp.full((16,), rr, jnp.int32), idx_all[c * ch + rr, :]],
                    zero)
            dmas[c] = pltpu.async_copy(buf, out_hbm.at[pl.ds(row0, ch)], sem)
        for c in range(nchunk - nbuf, nchunk):
            dmas[c].wait()

    return sc_scatter


@jax.jit
def kernel(tokens):
    b, l, d = tokens.shape
    tm = l // 2

    body = functools.partial(_topk_idx_body, tm=tm, l=l, topk=min(TOPK, l),
                             season=SEASON)
    idx16 = pl.pallas_call(
        body,
        grid=(b, l // tm),
        in_specs=[pl.BlockSpec((1, l, d), lambda bi, i: (bi, 0, 0))],
        out_specs=pl.BlockSpec((1, tm, 16), lambda bi, i: (bi, i, 0)),
        out_shape=jax.ShapeDtypeStruct((b, l, 16), jnp.int32),
        compiler_params=pltpu.CompilerParams(
            dimension_semantics=("parallel", "parallel")),
    )(tokens)

    sc_scatter = _make_sc_scatter(b * l, l)
    neg_chunk = jnp.full((16, l), NEG_INF, dtype=jnp.float32)
    out = sc_scatter(neg_chunk, idx16.reshape(b * l, 16))
    return out.reshape(b, 1, l, l)
